# baseline (speedup 1.0000x reference)
import functools
import math

import jax
import jax.numpy as jnp
from jax import lax
from jax.experimental import pallas as pl
from jax.experimental.pallas import tpu as pltpu
from jax.experimental.pallas import tpu_sc as plsc

_VOCAB = 100000
_MAXLEN = 2048
_HID = 128
_BATCH = 64

_INFO = plsc.get_sparse_core_info()
_NC = _INFO.num_cores
_NS = _INFO.num_subcores
_NW = _NC * _NS
_TPW = _MAXLEN // _NW
_LANES = _INFO.num_lanes
_SCALE = math.sqrt(float(_HID))
_K = 4
_NCHUNK = _BATCH // _K
_NSLOT = 3


def _body(x_hbm, tok_hbm, pos_hbm, out_hbm, idx_v, gbuf, pos_v, gsems, ssems):
    wid = lax.axis_index("s") * _NC + lax.axis_index("c")
    t0 = wid * _TPW
    c0 = (wid // 2) * (2 * _TPW)
    off = (wid % 2) * _TPW

    pltpu.sync_copy(pos_hbm.at[pl.ds(t0, _TPW)], pos_v)
    pltpu.sync_copy(x_hbm.at[:, pl.ds(c0, 2 * _TPW)], idx_v)

    def issue_gathers(c):
        s = c % _NSLOT
        for k in range(_K):
            b = c * _K + k
            pltpu.async_copy(tok_hbm.at[idx_v.at[b, pl.ds(off, _TPW)]],
                             gbuf.at[s, k], gsems[s])

    def wait_gathers(c):
        s = c % _NSLOT
        for k in range(_K):
            b = c * _K + k
            pltpu.make_async_copy(tok_hbm.at[idx_v.at[b, pl.ds(off, _TPW)]],
                                  gbuf.at[s, k], gsems[s]).wait()

    def store_copy(c):
        s = c % _NSLOT
        return pltpu.make_async_copy(
            gbuf.at[s], out_hbm.at[pl.ds(c * _K, _K), pl.ds(t0, _TPW)],
            ssems[s])

    def fma(c):
        s = c % _NSLOT

        def fma_row(r, carry):
            pv = [pos_v[r, pl.ds(j * _LANES, _LANES)]
                  for j in range(_HID // _LANES)]
            for k in range(_K):
                for j in range(_HID // _LANES):
                    sl = pl.ds(j * _LANES, _LANES)
                    gbuf[s, k, r, sl] = gbuf[s, k, r, sl] * _SCALE + pv[j]
            return carry

        lax.fori_loop(0, _TPW, fma_row, 0)

    issue_gathers(0)
    issue_gathers(1)
    for c in range(_NCHUNK):
        wait_gathers(c)
        fma(c)
        store_copy(c).start()
        if c + 2 < _NCHUNK:
            if c >= 1:
                store_copy(c - 1).wait()
            issue_gathers(c + 2)
    for c in range(_NCHUNK - _NSLOT, _NCHUNK):
        store_copy(c).wait()


@jax.jit
def kernel(x, token_table, pos_table):
    x = x.astype(jnp.int32)
    mesh = plsc.VectorSubcoreMesh(core_axis_name="c", subcore_axis_name="s")
    f = functools.partial(
        pl.kernel,
        mesh=mesh,
        out_type=jax.ShapeDtypeStruct((_BATCH, _MAXLEN, _HID), jnp.float32),
        scratch_types=[
            pltpu.VMEM((_BATCH, 2 * _TPW), jnp.int32),
            pltpu.VMEM((_NSLOT, _K, _TPW, _HID), jnp.float32),
            pltpu.VMEM((_TPW, _HID), jnp.float32),
            [pltpu.SemaphoreType.DMA] * _NSLOT,
            [pltpu.SemaphoreType.DMA] * _NSLOT,
        ],
    )(_body)
    return f(x, token_table, pos_table)
